# Initial kernel scaffold; baseline (speedup 1.0000x reference)
import functools

import jax
import jax.numpy as jnp
from jax import lax
from jax.experimental import pallas as pl
from jax.experimental.pallas import tpu as pltpu
from jax.experimental.pallas import tpu_sc as plsc

_N = 10000
_E = 320000
_D_IN = 128
_LANES = 16
_NTILES = 16
_N_PAD = 10240
_SLICE = _N_PAD // _NTILES
_E_PER = _E // _NTILES


def _rsqrt_nr(d):
  i = plsc.bitcast(d, jnp.int32)
  i = jnp.int32(0x5F3759DF) - lax.shift_right_arithmetic(i, jnp.int32(1))
  y = plsc.bitcast(i, jnp.float32)
  for _ in range(3):
    y = y * (1.5 - 0.5 * d * y * y)
  return y


def _mm_body(x_ref, w_ref, o_ref):
  o_ref[...] = jnp.dot(x_ref[...], w_ref[...],
                       preferred_element_type=jnp.float32)


def _sc_body(edge_hbm, h_hbm, b_hbm, out_hbm,
             srcv, dstv, tabp, acc, hfull, diss, ps, tmp2d, outs, bvec,
             shared_d, shared_p):
  w = lax.axis_index("s")
  base_n = w * _SLICE
  base_e = w * _E_PER

  pltpu.sync_copy(edge_hbm.at[0, pl.ds(base_e, _E_PER)], srcv)
  pltpu.sync_copy(edge_hbm.at[1, pl.ds(base_e, _E_PER)], dstv)
  pltpu.sync_copy(h_hbm, hfull.at[pl.ds(0, _N)])
  pltpu.sync_copy(b_hbm, bvec)

  zeros16 = jnp.zeros((_LANES,), jnp.float32)
  ones16 = jnp.ones((_LANES,), jnp.float32)

  def zero_body(k, c):
    i = k * _LANES
    tabp[pl.ds(i, _LANES)] = zeros16
    acc[pl.ds(i, _LANES)] = zeros16
    return c
  lax.fori_loop(0, _N_PAD // _LANES, zero_body, 0)

  def hist_body(k, c):
    i = k * _LANES
    d_idx = dstv[pl.ds(i, _LANES)]
    plsc.addupdate_scatter(tabp, [d_idx], ones16)
    return c
  lax.fori_loop(0, _E_PER // _LANES, hist_body, 0)

  pltpu.sync_copy(tabp, shared_d.at[w])
  plsc.subcore_barrier()

  pltpu.sync_copy(shared_d.at[:, pl.ds(base_n, _SLICE)], tmp2d)

  def degp_body(k, c):
    i = k * _LANES
    s = tmp2d[0, pl.ds(i, _LANES)]
    for t in range(1, _NTILES):
      s = s + tmp2d[t, pl.ds(i, _LANES)]
    y = _rsqrt_nr(s + 1.0)
    diss[pl.ds(i, _LANES)] = y
    ps[pl.ds(i, _LANES)] = hfull[pl.ds(base_n + i, _LANES)] * y
    return c
  lax.fori_loop(0, _SLICE // _LANES, degp_body, 0)

  pltpu.sync_copy(ps, shared_p.at[pl.ds(base_n, _SLICE)])
  plsc.subcore_barrier()
  pltpu.sync_copy(shared_p, tabp)

  def edge_body(k, c):
    i = k * _LANES
    sv = srcv[pl.ds(i, _LANES)]
    vals = plsc.load_gather(tabp, [sv])
    dv = dstv[pl.ds(i, _LANES)]
    plsc.addupdate_scatter(acc, [dv], vals)
    return c
  lax.fori_loop(0, _E_PER // _LANES, edge_body, 0)

  pltpu.sync_copy(acc, shared_d.at[w])
  plsc.subcore_barrier()

  pltpu.sync_copy(shared_d.at[:, pl.ds(base_n, _SLICE)], tmp2d)
  bv = bvec[pl.ds(0, _LANES)]

  def out_body(k, c):
    i = k * _LANES
    s = tmp2d[0, pl.ds(i, _LANES)]
    for t in range(1, _NTILES):
      s = s + tmp2d[t, pl.ds(i, _LANES)]
    o = diss[pl.ds(i, _LANES)] * (s + ps[pl.ds(i, _LANES)]) + bv
    outs[pl.ds(i, _LANES)] = o
    return c
  lax.fori_loop(0, _SLICE // _LANES, out_body, 0)

  pltpu.sync_copy(outs, out_hbm.at[pl.ds(base_n, _SLICE)])


def kernel(x, edge_index, W, b):
  n = x.shape[0]

  h = pl.pallas_call(
      _mm_body,
      grid=(10,),
      in_specs=[
          pl.BlockSpec((n // 10, _D_IN), lambda i: (i, 0)),
          pl.BlockSpec((_D_IN, 1), lambda i: (0, 0)),
      ],
      out_specs=pl.BlockSpec((n // 10, 1), lambda i: (i, 0)),
      out_shape=jax.ShapeDtypeStruct((n, 1), jnp.float32),
  )(x, W)
  h_flat = h.reshape(n)
  b16 = jnp.broadcast_to(b, (_LANES,)).astype(jnp.float32)

  mesh = plsc.VectorSubcoreMesh(core_axis_name="c", subcore_axis_name="s",
                                num_cores=1)
  sc_fn = pl.kernel(
      _sc_body,
      out_type=jax.ShapeDtypeStruct((_N_PAD,), jnp.float32),
      mesh=mesh,
      scratch_types=[
          pltpu.VMEM((_E_PER,), jnp.int32),
          pltpu.VMEM((_E_PER,), jnp.int32),
          pltpu.VMEM((_N_PAD,), jnp.float32),
          pltpu.VMEM((_N_PAD,), jnp.float32),
          pltpu.VMEM((_N_PAD,), jnp.float32),
          pltpu.VMEM((_SLICE,), jnp.float32),
          pltpu.VMEM((_SLICE,), jnp.float32),
          pltpu.VMEM((_NTILES, _SLICE), jnp.float32),
          pltpu.VMEM((_SLICE,), jnp.float32),
          pltpu.VMEM((_LANES,), jnp.float32),
          pltpu.VMEM_SHARED((_NTILES, _N_PAD), jnp.float32),
          pltpu.VMEM_SHARED((_N_PAD,), jnp.float32),
      ],
  )
  out_pad = sc_fn(edge_index, h_flat, b16)
  return out_pad[:n].reshape(n, 1)


# speedup vs baseline: 105.9472x; 105.9472x over previous
import functools

import jax
import jax.numpy as jnp
from jax import lax
from jax.experimental import pallas as pl
from jax.experimental.pallas import tpu as pltpu
from jax.experimental.pallas import tpu_sc as plsc

_N = 10000
_E = 320000
_D_IN = 128
_LANES = 16
_NTILES = 16
_N_PAD = 10240
_SLICE = _N_PAD // _NTILES
_E_PER = _E // _NTILES


def _rsqrt_nr(d):
  i = plsc.bitcast(d, jnp.int32)
  i = jnp.int32(0x5F3759DF) - lax.shift_right_arithmetic(i, jnp.int32(1))
  y = plsc.bitcast(i, jnp.float32)
  for _ in range(3):
    y = y * (1.5 - 0.5 * d * y * y)
  return y


def _mm_body(x_ref, w_ref, o_ref):
  o_ref[...] = jnp.dot(x_ref[...], w_ref[...],
                       preferred_element_type=jnp.float32)


def _sc_body(edge_hbm, h_hbm, b_hbm, out_hbm,
             srcv, dstv, tabp, acc, hfull, diss, ps, tmp2d, outs, bvec,
             shared_d, shared_p):
  w = lax.axis_index("s")
  base_n = w * _SLICE
  base_e = w * _E_PER

  pltpu.sync_copy(edge_hbm.at[pl.ds(base_e, _E_PER)], srcv)
  pltpu.sync_copy(edge_hbm.at[pl.ds(_E + base_e, _E_PER)], dstv)
  pltpu.sync_copy(h_hbm, hfull.at[pl.ds(0, _N)])
  pltpu.sync_copy(b_hbm, bvec)

  zeros16 = jnp.zeros((_LANES,), jnp.float32)
  ones16 = jnp.ones((_LANES,), jnp.float32)

  def zero_body(k, c):
    i = k * _LANES
    tabp[pl.ds(i, _LANES)] = zeros16
    acc[pl.ds(i, _LANES)] = zeros16
    return c
  lax.fori_loop(0, _N_PAD // _LANES, zero_body, 0)

  def hist_body(k, c):
    i = k * _LANES
    d_idx = dstv[pl.ds(i, _LANES)]
    plsc.addupdate_scatter(tabp, [d_idx], ones16)
    return c
  lax.fori_loop(0, _E_PER // _LANES, hist_body, 0)

  pltpu.sync_copy(tabp, shared_d.at[w])
  plsc.subcore_barrier()

  pltpu.sync_copy(shared_d.at[:, pl.ds(base_n, _SLICE)], tmp2d)

  def degp_body(k, c):
    i = k * _LANES
    s = tmp2d[0, pl.ds(i, _LANES)]
    for t in range(1, _NTILES):
      s = s + tmp2d[t, pl.ds(i, _LANES)]
    y = _rsqrt_nr(s + 1.0)
    diss[pl.ds(i, _LANES)] = y
    ps[pl.ds(i, _LANES)] = hfull[pl.ds(base_n + i, _LANES)] * y
    return c
  lax.fori_loop(0, _SLICE // _LANES, degp_body, 0)

  pltpu.sync_copy(ps, shared_p.at[pl.ds(base_n, _SLICE)])
  plsc.subcore_barrier()
  pltpu.sync_copy(shared_p, tabp)

  def edge_body(k, c):
    i = k * _LANES
    sv = srcv[pl.ds(i, _LANES)]
    vals = plsc.load_gather(tabp, [sv])
    dv = dstv[pl.ds(i, _LANES)]
    plsc.addupdate_scatter(acc, [dv], vals)
    return c
  lax.fori_loop(0, _E_PER // _LANES, edge_body, 0)

  pltpu.sync_copy(acc, shared_d.at[w])
  plsc.subcore_barrier()

  pltpu.sync_copy(shared_d.at[:, pl.ds(base_n, _SLICE)], tmp2d)
  bv = bvec[pl.ds(0, _LANES)]

  def out_body(k, c):
    i = k * _LANES
    s = tmp2d[0, pl.ds(i, _LANES)]
    for t in range(1, _NTILES):
      s = s + tmp2d[t, pl.ds(i, _LANES)]
    o = diss[pl.ds(i, _LANES)] * (s + ps[pl.ds(i, _LANES)]) + bv
    outs[pl.ds(i, _LANES)] = o
    return c
  lax.fori_loop(0, _SLICE // _LANES, out_body, 0)

  pltpu.sync_copy(outs, out_hbm.at[pl.ds(base_n, _SLICE)])


def kernel(x, edge_index, W, b):
  n = x.shape[0]

  h = pl.pallas_call(
      _mm_body,
      grid=(10,),
      in_specs=[
          pl.BlockSpec((n // 10, _D_IN), lambda i: (i, 0)),
          pl.BlockSpec((_D_IN, 1), lambda i: (0, 0)),
      ],
      out_specs=pl.BlockSpec((n // 10, 1), lambda i: (i, 0)),
      out_shape=jax.ShapeDtypeStruct((n, 1), jnp.float32),
  )(x, W)
  h_flat = h.reshape(n)
  b16 = jnp.broadcast_to(b, (_LANES,)).astype(jnp.float32)

  mesh = plsc.VectorSubcoreMesh(core_axis_name="c", subcore_axis_name="s",
                                num_cores=1)
  sc_fn = pl.kernel(
      _sc_body,
      out_type=jax.ShapeDtypeStruct((_N_PAD,), jnp.float32),
      mesh=mesh,
      compiler_params=pltpu.CompilerParams(needs_layout_passes=False),
      scratch_types=[
          pltpu.VMEM((_E_PER,), jnp.int32),
          pltpu.VMEM((_E_PER,), jnp.int32),
          pltpu.VMEM((_N_PAD,), jnp.float32),
          pltpu.VMEM((_N_PAD,), jnp.float32),
          pltpu.VMEM((_N_PAD,), jnp.float32),
          pltpu.VMEM((_SLICE,), jnp.float32),
          pltpu.VMEM((_SLICE,), jnp.float32),
          pltpu.VMEM((_NTILES, _SLICE), jnp.float32),
          pltpu.VMEM((_SLICE,), jnp.float32),
          pltpu.VMEM((_LANES,), jnp.float32),
          pltpu.VMEM_SHARED((_NTILES, _N_PAD), jnp.float32),
          pltpu.VMEM_SHARED((_N_PAD,), jnp.float32),
      ],
  )
  out_pad = sc_fn(edge_index.reshape(-1), h_flat, b16)
  return out_pad[:n].reshape(n, 1)


# speedup vs baseline: 138.0290x; 1.3028x over previous
import functools

import jax
import jax.numpy as jnp
from jax import lax
from jax.experimental import pallas as pl
from jax.experimental.pallas import tpu as pltpu
from jax.experimental.pallas import tpu_sc as plsc

_N = 10000
_E = 320000
_D_IN = 128
_LANES = 16
_NTILES = 16
_N_PAD = 10240
_SLICE = _N_PAD // _NTILES
_E_PER = _E // _NTILES


def _rsqrt_nr(d):
  i = plsc.bitcast(d, jnp.int32)
  i = jnp.int32(0x5F3759DF) - lax.shift_right_arithmetic(i, jnp.int32(1))
  y = plsc.bitcast(i, jnp.float32)
  for _ in range(3):
    y = y * (1.5 - 0.5 * d * y * y)
  return y


def _mm_body(x_ref, w_ref, o_ref):
  o_ref[...] = jnp.dot(x_ref[...], w_ref[...],
                       preferred_element_type=jnp.float32)


def _sc_body(edge_hbm, h_hbm, b_hbm, out_hbm,
             srcv, dstv, tabp, acc, hfull, diss, ps, tmp2d, outs, bvec,
             shared_d, shared_p):
  w = lax.axis_index("s")
  base_n = w * _SLICE
  base_e = w * _E_PER

  pltpu.sync_copy(edge_hbm.at[pl.ds(base_e, _E_PER)], srcv)
  pltpu.sync_copy(edge_hbm.at[pl.ds(_E + base_e, _E_PER)], dstv)
  pltpu.sync_copy(h_hbm, hfull.at[pl.ds(0, _N)])
  pltpu.sync_copy(b_hbm, bvec)

  zeros16 = jnp.zeros((_LANES,), jnp.float32)
  ones16 = jnp.ones((_LANES,), jnp.float32)

  @plsc.parallel_loop(0, _N_PAD, step=_LANES, unroll=8)
  def zero_body(i):
    tabp[pl.ds(i, _LANES)] = zeros16
    acc[pl.ds(i, _LANES)] = zeros16

  @plsc.parallel_loop(0, _E_PER, step=_LANES, unroll=8)
  def hist_body(i):
    d_idx = dstv[pl.ds(i, _LANES)]
    plsc.addupdate_scatter(tabp, [d_idx], ones16)

  pltpu.sync_copy(tabp, shared_d.at[w])
  plsc.subcore_barrier()

  pltpu.sync_copy(shared_d.at[:, pl.ds(base_n, _SLICE)], tmp2d)

  @plsc.parallel_loop(0, _SLICE, step=_LANES, unroll=2)
  def degp_body(i):
    s = tmp2d[0, pl.ds(i, _LANES)]
    for t in range(1, _NTILES):
      s = s + tmp2d[t, pl.ds(i, _LANES)]
    y = _rsqrt_nr(s + 1.0)
    diss[pl.ds(i, _LANES)] = y
    ps[pl.ds(i, _LANES)] = hfull[pl.ds(base_n + i, _LANES)] * y

  pltpu.sync_copy(ps, shared_p.at[pl.ds(base_n, _SLICE)])
  plsc.subcore_barrier()
  pltpu.sync_copy(shared_p, tabp)

  @plsc.parallel_loop(0, _E_PER, step=_LANES, unroll=8)
  def edge_body(i):
    sv = srcv[pl.ds(i, _LANES)]
    vals = plsc.load_gather(tabp, [sv])
    dv = dstv[pl.ds(i, _LANES)]
    plsc.addupdate_scatter(acc, [dv], vals)

  pltpu.sync_copy(acc, shared_d.at[w])
  plsc.subcore_barrier()

  pltpu.sync_copy(shared_d.at[:, pl.ds(base_n, _SLICE)], tmp2d)
  bv = bvec[pl.ds(0, _LANES)]

  @plsc.parallel_loop(0, _SLICE, step=_LANES, unroll=2)
  def out_body(i):
    s = tmp2d[0, pl.ds(i, _LANES)]
    for t in range(1, _NTILES):
      s = s + tmp2d[t, pl.ds(i, _LANES)]
    o = diss[pl.ds(i, _LANES)] * (s + ps[pl.ds(i, _LANES)]) + bv
    outs[pl.ds(i, _LANES)] = o

  pltpu.sync_copy(outs, out_hbm.at[pl.ds(base_n, _SLICE)])


def kernel(x, edge_index, W, b):
  n = x.shape[0]

  h = pl.pallas_call(
      _mm_body,
      grid=(10,),
      in_specs=[
          pl.BlockSpec((n // 10, _D_IN), lambda i: (i, 0)),
          pl.BlockSpec((_D_IN, 1), lambda i: (0, 0)),
      ],
      out_specs=pl.BlockSpec((n // 10, 1), lambda i: (i, 0)),
      out_shape=jax.ShapeDtypeStruct((n, 1), jnp.float32),
  )(x, W)
  h_flat = h.reshape(n)
  b16 = jnp.broadcast_to(b, (_LANES,)).astype(jnp.float32)

  mesh = plsc.VectorSubcoreMesh(core_axis_name="c", subcore_axis_name="s",
                                num_cores=1)
  sc_fn = pl.kernel(
      _sc_body,
      out_type=jax.ShapeDtypeStruct((_N_PAD,), jnp.float32),
      mesh=mesh,
      compiler_params=pltpu.CompilerParams(needs_layout_passes=False),
      scratch_types=[
          pltpu.VMEM((_E_PER,), jnp.int32),
          pltpu.VMEM((_E_PER,), jnp.int32),
          pltpu.VMEM((_N_PAD,), jnp.float32),
          pltpu.VMEM((_N_PAD,), jnp.float32),
          pltpu.VMEM((_N_PAD,), jnp.float32),
          pltpu.VMEM((_SLICE,), jnp.float32),
          pltpu.VMEM((_SLICE,), jnp.float32),
          pltpu.VMEM((_NTILES, _SLICE), jnp.float32),
          pltpu.VMEM((_SLICE,), jnp.float32),
          pltpu.VMEM((_LANES,), jnp.float32),
          pltpu.VMEM_SHARED((_NTILES, _N_PAD), jnp.float32),
          pltpu.VMEM_SHARED((_N_PAD,), jnp.float32),
      ],
  )
  out_pad = sc_fn(edge_index.reshape(-1), h_flat, b16)
  return out_pad[:n].reshape(n, 1)
